# baseline (speedup 1.0000x reference)
import jax
import jax.numpy as jnp
from jax import lax
from jax.experimental import pallas as pl
from jax.experimental.pallas import tpu as pltpu

_HEAD_DIM = 128
_ROPE_THETA = 10000.0
_G = 8


def _rope_tc(S):
    inv_freq = 1.0 / (
        _ROPE_THETA ** (jnp.arange(0, _HEAD_DIM, 2, dtype=jnp.float32) / _HEAD_DIM)
    )
    inv2 = jnp.concatenate([inv_freq, inv_freq]).reshape(1, _HEAD_DIM)

    def body(inv_ref, out_ref):
        pos = lax.broadcasted_iota(jnp.int32, (S, _HEAD_DIM), 0).astype(jnp.float32)
        freqs = pos * inv_ref[0, :]
        out_ref[0] = jnp.cos(freqs)
        out_ref[1] = jnp.sin(freqs)

    return pl.pallas_call(
        body,
        out_shape=jax.ShapeDtypeStruct((2, S, _HEAD_DIM), jnp.float32),
    )(inv2)


def _tc_gather(ids_flat, table):
    T = ids_flat.shape[0]
    D = table.shape[1]
    G = _G
    assert T % G == 0

    V = table.shape[0]
    SL = D // 128
    table3 = table.reshape(V, SL, 128)

    def body(ids_ref, *refs):
        in_refs = refs[:G]
        out_ref = refs[G]
        for g in range(G):
            out_ref[g] = in_refs[g][0]

    grid_spec = pltpu.PrefetchScalarGridSpec(
        num_scalar_prefetch=1,
        grid=(T // G,),
        in_specs=[
            pl.BlockSpec((1, SL, 128), (lambda i, ids, g=g: (ids[i * G + g], 0, 0)))
            for g in range(G)
        ],
        out_specs=pl.BlockSpec((G, SL, 128), lambda i, ids: (i, 0, 0)),
    )
    out = pl.pallas_call(
        body,
        grid_spec=grid_spec,
        out_shape=jax.ShapeDtypeStruct((T, SL, 128), jnp.float32),
    )(ids_flat, *([table3] * G))
    return out.reshape(T, D)


def kernel(input_ids, attention_mask, table):
    B, S = input_ids.shape
    D = table.shape[1]
    N = B * S
    hidden = _tc_gather(input_ids.reshape(N), table).reshape(B, S, D)
    position_embeddings = _rope_tc(S)[:, None]
    return (hidden, attention_mask, position_embeddings)
